# baseline (speedup 1.0000x reference)
import functools

import jax
import jax.numpy as jnp
from jax import lax
from jax.experimental import pallas as pl
from jax.experimental.pallas import tpu as pltpu
from jax.experimental.pallas import tpu_sc as plsc

_SC_PARAMS = pltpu.CompilerParams(use_tc_tiling_on_sc=False)

NC = 2
NS = 16
NW = NC * NS
L = 16

N_PAD = 10240
CHUNK = 128
ZROWS = N_PAD // NS
D = 128


def _mm_body(x_ref, w_ref, o_ref):
    o_ref[...] = jnp.dot(x_ref[...], w_ref[...],
                         preferred_element_type=jnp.float32)


def _matmul(x, w):
    n = x.shape[0]
    blk = 2048
    return pl.pallas_call(
        _mm_body,
        grid=(n // blk,),
        in_specs=[pl.BlockSpec((blk, D), lambda i: (i, 0)),
                  pl.BlockSpec((D, D), lambda i: (0, 0))],
        out_specs=pl.BlockSpec((blk, D), lambda i: (i, 0)),
        out_shape=jax.ShapeDtypeStruct((n, D), jnp.float32),
    )(x, w)


def _fuse1_body(x_ref, ws_ref, b_ref, agg_ref, deg_ref, wn_ref, h_ref, z_ref):
    deg = deg_ref[...][:, 0:1]
    inv = 1.0 / jnp.maximum(deg, 1.0)
    agg = agg_ref[...] * inv
    h = jnp.dot(x_ref[...], ws_ref[...], preferred_element_type=jnp.float32)
    h = jnp.maximum(h + agg + b_ref[...], 0.0)
    h_ref[...] = h
    z_ref[...] = jnp.dot(h, wn_ref[...], preferred_element_type=jnp.float32)


def _fuse1(x, w_self, b, agg, deg, w_neigh_next):
    n = x.shape[0]
    blk = 2048
    return pl.pallas_call(
        _fuse1_body,
        grid=(n // blk,),
        in_specs=[pl.BlockSpec((blk, D), lambda i: (i, 0)),
                  pl.BlockSpec((D, D), lambda i: (0, 0)),
                  pl.BlockSpec((1, D), lambda i: (0, 0)),
                  pl.BlockSpec((blk, D), lambda i: (i, 0)),
                  pl.BlockSpec((blk, L), lambda i: (i, 0)),
                  pl.BlockSpec((D, D), lambda i: (0, 0))],
        out_specs=[pl.BlockSpec((blk, D), lambda i: (i, 0)),
                   pl.BlockSpec((blk, D), lambda i: (i, 0))],
        out_shape=[jax.ShapeDtypeStruct((n, D), jnp.float32),
                   jax.ShapeDtypeStruct((n, D), jnp.float32)],
    )(x, w_self, b, agg, deg, w_neigh_next)


def _fuse2_body(x_ref, ws_ref, b_ref, agg_ref, deg_ref, h_ref):
    deg = deg_ref[...][:, 0:1]
    inv = 1.0 / jnp.maximum(deg, 1.0)
    agg = agg_ref[...] * inv
    h = jnp.dot(x_ref[...], ws_ref[...], preferred_element_type=jnp.float32)
    h_ref[...] = h + agg + b_ref[...]


def _fuse2(x, w_self, b, agg, deg):
    n = x.shape[0]
    blk = 2048
    return pl.pallas_call(
        _fuse2_body,
        grid=(n // blk,),
        in_specs=[pl.BlockSpec((blk, D), lambda i: (i, 0)),
                  pl.BlockSpec((D, D), lambda i: (0, 0)),
                  pl.BlockSpec((1, D), lambda i: (0, 0)),
                  pl.BlockSpec((blk, D), lambda i: (i, 0)),
                  pl.BlockSpec((blk, L), lambda i: (i, 0))],
        out_specs=pl.BlockSpec((blk, D), lambda i: (i, 0)),
        out_shape=jax.ShapeDtypeStruct((n, D), jnp.float32),
    )(x, w_self, b, agg, deg)


def _reduce_body(p_ref, o_ref):
    o_ref[...] = jnp.sum(p_ref[...], axis=1, keepdims=True)


def _reduce_partials(p):
    n = p.shape[0]
    blk = 4096
    return pl.pallas_call(
        _reduce_body,
        grid=(n // blk,),
        in_specs=[pl.BlockSpec((blk, L), lambda i: (i, 0))],
        out_specs=pl.BlockSpec((blk, 1), lambda i: (i, 0)),
        out_shape=jax.ShapeDtypeStruct((n, 1), jnp.float32),
    )(p)


def _zero_vmem_rows(ref, nrows, ncol16):
    zv = jnp.zeros((L,), jnp.float32)

    def row(r, _):
        for j in range(ncol16):
            ref[r, pl.ds(j * L, L)] = zv
        return 0

    lax.fori_loop(0, nrows, row, 0)


def _segsum_sc(z, src2d, dst2d, with_deg):
    del with_deg
    cpt = src2d.shape[0] // NS
    nph = 5
    cpp = cpt // nph

    out_type = [jax.ShapeDtypeStruct((N_PAD, D), jnp.float32)]
    scratch = [
        pltpu.VMEM((cpp, CHUNK), jnp.int32),
        pltpu.VMEM((cpp, CHUNK), jnp.int32),
        pltpu.VMEM((CHUNK, D), jnp.float32),
        pltpu.VMEM((CHUNK, D), jnp.float32),
        pltpu.VMEM_SHARED((N_PAD, D), jnp.float32),
        pltpu.SemaphoreType.DMA,
        pltpu.SemaphoreType.DMA,
    ]

    def body(z_hbm, src_hbm, dst_hbm, agg_out,
             src_v, dst_v, rows0, rows1, tab, sem_g, sem_s):
        c = lax.axis_index("c")
        s = lax.axis_index("s")
        rows = [rows0, rows1]

        @pl.when(c == 0)
        def _zero():
            _zero_vmem_rows(rows0, CHUNK, D // L)
            for k in range(ZROWS // CHUNK):
                pltpu.sync_copy(rows0,
                                tab.at[pl.ds(s * ZROWS + k * CHUNK, CHUNK)])

        plsc.subcore_barrier()

        @pl.when(c == 0)
        def _run():
            for p in range(nph):
                off = pl.multiple_of(s * cpt + p * cpp, 8)
                pltpu.sync_copy(src_hbm.at[pl.ds(off, cpp)], src_v)
                pltpu.sync_copy(dst_hbm.at[pl.ds(off, cpp)], dst_v)
                pend_s = [None, None]
                pend_g = pltpu.async_copy(z_hbm.at[src_v.at[0]], rows[0],
                                          sem_g)
                for j in range(cpp):
                    b = j & 1
                    nb = b ^ 1
                    pend_g.wait()
                    if j + 1 < cpp:
                        if pend_s[nb] is not None:
                            pend_s[nb].wait()
                            pend_s[nb] = None
                        pend_g = pltpu.async_copy(
                            z_hbm.at[src_v.at[j + 1]], rows[nb], sem_g)
                    pend_s[b] = pltpu.async_copy(
                        rows[b], tab.at[dst_v.at[j]], sem_s, add=True)
                for b in range(2):
                    if pend_s[b] is not None:
                        pend_s[b].wait()

        plsc.subcore_barrier()

        @pl.when(c == 0)
        def _dump():
            pltpu.sync_copy(tab.at[pl.ds(s * ZROWS, ZROWS)],
                            agg_out.at[pl.ds(s * ZROWS, ZROWS)])

    mesh = plsc.VectorSubcoreMesh(core_axis_name="c", subcore_axis_name="s")
    fn = pl.kernel(body, out_type=out_type, mesh=mesh, scratch_types=scratch,
                   compiler_params=_SC_PARAMS)
    return fn(z, src2d, dst2d)[0]


def _deg_sc(dst2d):
    cpt = dst2d.shape[0] // NS

    scratch = [
        pltpu.VMEM((cpt, CHUNK), jnp.int32),
        pltpu.VMEM((CHUNK, L), jnp.float32),
        pltpu.VMEM_SHARED((N_PAD, L), jnp.float32),
        pltpu.SemaphoreType.DMA,
    ]

    def body(dst_hbm, deg_out, dst_v, ones_v, degtab, sem):
        c = lax.axis_index("c")
        s = lax.axis_index("s")

        @pl.when(c == 0)
        def _run():
            pltpu.sync_copy(dst_hbm.at[pl.ds(s * cpt, cpt)], dst_v)
            _zero_vmem_rows(ones_v, CHUNK, 1)
            for k in range(ZROWS // CHUNK):
                pltpu.sync_copy(ones_v,
                                degtab.at[pl.ds(s * ZROWS + k * CHUNK, CHUNK)])

        plsc.subcore_barrier()

        @pl.when(c == 0)
        def _scatter():
            ones = jnp.full((L,), 1.0, jnp.float32)

            def fill(r, _):
                ones_v[r, :] = ones
                return 0

            lax.fori_loop(0, CHUNK, fill, 0)
            pend = []
            for j in range(cpt):
                if len(pend) >= 8:
                    pend.pop(0).wait()
                pend.append(pltpu.async_copy(
                    ones_v, degtab.at[dst_v.at[j]], sem, add=True))
            for d in pend:
                d.wait()

        plsc.subcore_barrier()

        @pl.when(c == 0)
        def _dump():
            pltpu.sync_copy(degtab.at[pl.ds(s * ZROWS, ZROWS)],
                            deg_out.at[pl.ds(s * ZROWS, ZROWS)])

    mesh = plsc.VectorSubcoreMesh(core_axis_name="c", subcore_axis_name="s")
    fn = pl.kernel(
        body,
        out_type=jax.ShapeDtypeStruct((N_PAD, L), jnp.float32),
        mesh=mesh,
        scratch_types=scratch,
        compiler_params=_SC_PARAMS,
    )
    return fn(dst2d)


def _decode_sc(h2, u1, v1):
    ept = u1.shape[0] // NS
    cpt = ept // CHUNK

    scratch = [
        pltpu.VMEM((ept,), jnp.int32),
        pltpu.VMEM((ept,), jnp.int32),
        pltpu.VMEM((CHUNK, D), jnp.float32),
        pltpu.VMEM((CHUNK, D), jnp.float32),
        pltpu.VMEM((CHUNK, D), jnp.float32),
        pltpu.VMEM((CHUNK, D), jnp.float32),
        pltpu.VMEM((CHUNK, L), jnp.float32),
        pltpu.VMEM((CHUNK, L), jnp.float32),
        pltpu.SemaphoreType.DMA,
        pltpu.SemaphoreType.DMA,
    ]

    def body(h_hbm, u_hbm, v_hbm, out_hbm,
             u_v, v_v, ur0, ur1, vr0, vr1, ob0, ob1, sem, sem_o):
        c = lax.axis_index("c")
        s = lax.axis_index("s")
        ur = [ur0, ur1]
        vr = [vr0, vr1]
        ob = [ob0, ob1]

        @pl.when(c == 0)
        def _run():
            pltpu.sync_copy(u_hbm.at[pl.ds(s * ept, ept)], u_v)
            pltpu.sync_copy(v_hbm.at[pl.ds(s * ept, ept)], v_v)

            def issue(j, b):
                return (
                    pltpu.async_copy(
                        h_hbm.at[u_v.at[pl.ds(j * CHUNK, CHUNK)]], ur[b], sem),
                    pltpu.async_copy(
                        h_hbm.at[v_v.at[pl.ds(j * CHUNK, CHUNK)]], vr[b], sem),
                )

            pend = [None, None]
            pend_o = [None, None]
            pend[0] = issue(0, 0)
            for j in range(cpt):
                b = j & 1
                nb = b ^ 1
                gu, gv = pend[b]
                gu.wait()
                gv.wait()
                if j + 1 < cpt:
                    pend[nb] = issue(j + 1, nb)
                if pend_o[b] is not None:
                    pend_o[b].wait()
                    pend_o[b] = None
                urb = ur[b]
                vrb = vr[b]
                obb = ob[b]

                def edge(e, _):
                    acc = urb[e, pl.ds(0, L)] * vrb[e, pl.ds(0, L)]
                    for k in range(1, D // L):
                        acc = acc + (urb[e, pl.ds(k * L, L)]
                                     * vrb[e, pl.ds(k * L, L)])
                    obb[e, :] = acc
                    return 0

                lax.fori_loop(0, CHUNK, edge, 0)
                pend_o[b] = pltpu.async_copy(
                    obb, out_hbm.at[pl.ds(s * ept + j * CHUNK, CHUNK)], sem_o)
            for b in range(2):
                if pend_o[b] is not None:
                    pend_o[b].wait()

    mesh = plsc.VectorSubcoreMesh(core_axis_name="c", subcore_axis_name="s")
    fn = pl.kernel(
        body,
        out_type=jax.ShapeDtypeStruct((u1.shape[0], L), jnp.float32),
        mesh=mesh,
        scratch_types=scratch,
        compiler_params=_SC_PARAMS,
    )
    return fn(h2, u1, v1)


def _pad_edges(idx, per_tile_chunks, reshape2d):
    total = NW * per_tile_chunks * CHUNK
    pad = total - idx.shape[0]
    idx = jnp.concatenate(
        [idx, jnp.full((pad,), N_PAD - 1, jnp.int32)]) if pad else idx
    return idx.reshape(NW * per_tile_chunks, CHUNK) if reshape2d else idx


def kernel(x, edge_index, decode_edge_index,
           W_self0, W_neigh0, b0, W_self1, W_neigh1, b1):
    n_nodes = x.shape[0]
    n_edges = edge_index.shape[1]
    n_dec = decode_edge_index.shape[1]

    e_cpt = -(-n_edges // (NW * CHUNK * 8)) * 8
    d_cpt = -(-n_dec // (NW * CHUNK))

    x_p = jnp.pad(x, ((0, N_PAD - n_nodes), (0, 0)))
    src2d = _pad_edges(edge_index[0], e_cpt, True)
    dst2d = _pad_edges(edge_index[1], e_cpt, True)
    u1 = _pad_edges(decode_edge_index[0], d_cpt, False)
    v1 = _pad_edges(decode_edge_index[1], d_cpt, False)

    b0r = b0.reshape(1, D)
    b1r = b1.reshape(1, D)

    z0 = _matmul(x_p, W_neigh0)
    deg = _deg_sc(dst2d)
    agg0 = _segsum_sc(z0, src2d, dst2d, with_deg=True)
    h1, z1 = _fuse1(x_p, W_self0, b0r, agg0, deg, W_neigh1)
    agg1 = _segsum_sc(z1, src2d, dst2d, with_deg=False)
    h2 = _fuse2(h1, W_self1, b1r, agg1, deg)
    parts = _decode_sc(h2, u1, v1)
    logits = _reduce_partials(parts)
    return logits[:n_dec]


# speedup vs baseline: 1.1548x; 1.1548x over previous
import functools

import jax
import jax.numpy as jnp
from jax import lax
from jax.experimental import pallas as pl
from jax.experimental.pallas import tpu as pltpu
from jax.experimental.pallas import tpu_sc as plsc

_SC_PARAMS = pltpu.CompilerParams(use_tc_tiling_on_sc=False)

NC = 2
NS = 16
NW = NC * NS
L = 16

N_PAD = 10240
CHUNK = 128
ZROWS = N_PAD // NS
D = 128


def _mm_body(x_ref, w_ref, o_ref):
    o_ref[...] = jnp.dot(x_ref[...], w_ref[...],
                         preferred_element_type=jnp.float32)


def _matmul(x, w):
    n = x.shape[0]
    blk = 2048
    return pl.pallas_call(
        _mm_body,
        grid=(n // blk,),
        in_specs=[pl.BlockSpec((blk, D), lambda i: (i, 0)),
                  pl.BlockSpec((D, D), lambda i: (0, 0))],
        out_specs=pl.BlockSpec((blk, D), lambda i: (i, 0)),
        out_shape=jax.ShapeDtypeStruct((n, D), jnp.float32),
    )(x, w)


def _fuse1_body(x_ref, ws_ref, b_ref, agg_ref, deg_ref, wn_ref, h_ref, z_ref):
    deg = deg_ref[...][:, 0:1]
    inv = 1.0 / jnp.maximum(deg, 1.0)
    a = agg_ref[...]
    agg = (a[0] + a[1]) * inv
    h = jnp.dot(x_ref[...], ws_ref[...], preferred_element_type=jnp.float32)
    h = jnp.maximum(h + agg + b_ref[...], 0.0)
    h_ref[...] = h
    z_ref[...] = jnp.dot(h, wn_ref[...], preferred_element_type=jnp.float32)


def _fuse1(x, w_self, b, agg, deg, w_neigh_next):
    n = x.shape[0]
    blk = 2048
    return pl.pallas_call(
        _fuse1_body,
        grid=(n // blk,),
        in_specs=[pl.BlockSpec((blk, D), lambda i: (i, 0)),
                  pl.BlockSpec((D, D), lambda i: (0, 0)),
                  pl.BlockSpec((1, D), lambda i: (0, 0)),
                  pl.BlockSpec((NC, blk, D), lambda i: (0, i, 0)),
                  pl.BlockSpec((blk, L), lambda i: (i, 0)),
                  pl.BlockSpec((D, D), lambda i: (0, 0))],
        out_specs=[pl.BlockSpec((blk, D), lambda i: (i, 0)),
                   pl.BlockSpec((blk, D), lambda i: (i, 0))],
        out_shape=[jax.ShapeDtypeStruct((n, D), jnp.float32),
                   jax.ShapeDtypeStruct((n, D), jnp.float32)],
    )(x, w_self, b, agg, deg, w_neigh_next)


def _fuse2_body(x_ref, ws_ref, b_ref, agg_ref, deg_ref, h_ref):
    deg = deg_ref[...][:, 0:1]
    inv = 1.0 / jnp.maximum(deg, 1.0)
    a = agg_ref[...]
    agg = (a[0] + a[1]) * inv
    h = jnp.dot(x_ref[...], ws_ref[...], preferred_element_type=jnp.float32)
    h_ref[...] = h + agg + b_ref[...]


def _fuse2(x, w_self, b, agg, deg):
    n = x.shape[0]
    blk = 2048
    return pl.pallas_call(
        _fuse2_body,
        grid=(n // blk,),
        in_specs=[pl.BlockSpec((blk, D), lambda i: (i, 0)),
                  pl.BlockSpec((D, D), lambda i: (0, 0)),
                  pl.BlockSpec((1, D), lambda i: (0, 0)),
                  pl.BlockSpec((NC, blk, D), lambda i: (0, i, 0)),
                  pl.BlockSpec((blk, L), lambda i: (i, 0))],
        out_specs=pl.BlockSpec((blk, D), lambda i: (i, 0)),
        out_shape=jax.ShapeDtypeStruct((n, D), jnp.float32),
    )(x, w_self, b, agg, deg)


def _reduce_body(p_ref, o_ref):
    o_ref[...] = jnp.sum(p_ref[...], axis=1, keepdims=True)


def _reduce_partials(p):
    n = p.shape[0]
    blk = 4096
    return pl.pallas_call(
        _reduce_body,
        grid=(n // blk,),
        in_specs=[pl.BlockSpec((blk, L), lambda i: (i, 0))],
        out_specs=pl.BlockSpec((blk, 1), lambda i: (i, 0)),
        out_shape=jax.ShapeDtypeStruct((n, 1), jnp.float32),
    )(p)


def _zero_vmem_rows(ref, nrows, ncol16):
    zv = jnp.zeros((L,), jnp.float32)

    def row(r, _):
        for j in range(ncol16):
            ref[r, pl.ds(j * L, L)] = zv
        return 0

    lax.fori_loop(0, nrows, row, 0)


def _segsum_sc(z, src2d, dst2d, with_deg):
    del with_deg
    cpt0, cpt1 = 96, 64
    cpp = 8

    out_type = [jax.ShapeDtypeStruct((NC, N_PAD, D), jnp.float32)]
    scratch = [
        pltpu.VMEM((cpp, CHUNK), jnp.int32),
        pltpu.VMEM((cpp, CHUNK), jnp.int32),
        pltpu.VMEM((CHUNK, D), jnp.float32),
        pltpu.VMEM((CHUNK, D), jnp.float32),
        pltpu.VMEM_SHARED((N_PAD, D), jnp.float32),
        pltpu.SemaphoreType.DMA,
        pltpu.SemaphoreType.DMA,
    ]

    def body(z_hbm, src_hbm, dst_hbm, agg_out,
             src_v, dst_v, rows0, rows1, tab, sem_g, sem_s):
        c = lax.axis_index("c")
        s = lax.axis_index("s")
        rows = [rows0, rows1]

        _zero_vmem_rows(rows0, CHUNK, D // L)
        for k in range(ZROWS // CHUNK):
            pltpu.sync_copy(rows0,
                            tab.at[pl.ds(s * ZROWS + k * CHUNK, CHUNK)])

        plsc.subcore_barrier()

        def run(base, cptc):
            for p in range(cptc // cpp):
                off = pl.multiple_of(base + p * cpp, 8)
                pltpu.sync_copy(src_hbm.at[pl.ds(off, cpp)], src_v)
                pltpu.sync_copy(dst_hbm.at[pl.ds(off, cpp)], dst_v)
                pend_s = [None, None]
                pend_g = pltpu.async_copy(z_hbm.at[src_v.at[0]], rows[0],
                                          sem_g)
                for j in range(cpp):
                    b = j & 1
                    nb = b ^ 1
                    pend_g.wait()
                    if j + 1 < cpp:
                        if pend_s[nb] is not None:
                            pend_s[nb].wait()
                            pend_s[nb] = None
                        pend_g = pltpu.async_copy(
                            z_hbm.at[src_v.at[j + 1]], rows[nb], sem_g)
                    pend_s[b] = pltpu.async_copy(
                        rows[b], tab.at[dst_v.at[j]], sem_s, add=True)
                for b in range(2):
                    if pend_s[b] is not None:
                        pend_s[b].wait()

        @pl.when(c == 0)
        def _run0():
            run(s * cpt0, cpt0)

        @pl.when(c == 1)
        def _run1():
            run(NS * cpt0 + s * cpt1, cpt1)

        plsc.subcore_barrier()

        pltpu.sync_copy(tab.at[pl.ds(s * ZROWS, ZROWS)],
                        agg_out.at[c, pl.ds(s * ZROWS, ZROWS)])

    mesh = plsc.VectorSubcoreMesh(core_axis_name="c", subcore_axis_name="s")
    fn = pl.kernel(body, out_type=out_type, mesh=mesh, scratch_types=scratch,
                   compiler_params=_SC_PARAMS)
    return fn(z, src2d, dst2d)[0]


def _deg_sc(dst2d):
    cpt = dst2d.shape[0] // NS

    scratch = [
        pltpu.VMEM((cpt, CHUNK), jnp.int32),
        pltpu.VMEM((CHUNK, L), jnp.float32),
        pltpu.VMEM_SHARED((N_PAD, L), jnp.float32),
        pltpu.SemaphoreType.DMA,
    ]

    def body(dst_hbm, deg_out, dst_v, ones_v, degtab, sem):
        c = lax.axis_index("c")
        s = lax.axis_index("s")

        @pl.when(c == 0)
        def _run():
            pltpu.sync_copy(dst_hbm.at[pl.ds(s * cpt, cpt)], dst_v)
            _zero_vmem_rows(ones_v, CHUNK, 1)
            for k in range(ZROWS // CHUNK):
                pltpu.sync_copy(ones_v,
                                degtab.at[pl.ds(s * ZROWS + k * CHUNK, CHUNK)])

        plsc.subcore_barrier()

        @pl.when(c == 0)
        def _scatter():
            ones = jnp.full((L,), 1.0, jnp.float32)

            def fill(r, _):
                ones_v[r, :] = ones
                return 0

            lax.fori_loop(0, CHUNK, fill, 0)
            pend = []
            for j in range(cpt):
                if len(pend) >= 8:
                    pend.pop(0).wait()
                pend.append(pltpu.async_copy(
                    ones_v, degtab.at[dst_v.at[j]], sem, add=True))
            for d in pend:
                d.wait()

        plsc.subcore_barrier()

        @pl.when(c == 0)
        def _dump():
            pltpu.sync_copy(degtab.at[pl.ds(s * ZROWS, ZROWS)],
                            deg_out.at[pl.ds(s * ZROWS, ZROWS)])

    mesh = plsc.VectorSubcoreMesh(core_axis_name="c", subcore_axis_name="s")
    fn = pl.kernel(
        body,
        out_type=jax.ShapeDtypeStruct((N_PAD, L), jnp.float32),
        mesh=mesh,
        scratch_types=scratch,
        compiler_params=_SC_PARAMS,
    )
    return fn(dst2d)


def _decode_sc(h2, u1, v1):
    cpt_all = u1.shape[0] // (NS * CHUNK)
    cpt0 = 32
    cpt1 = cpt_all - cpt0
    ept = max(cpt0, cpt1) * CHUNK

    scratch = [
        pltpu.VMEM((ept,), jnp.int32),
        pltpu.VMEM((ept,), jnp.int32),
        pltpu.VMEM((CHUNK, D), jnp.float32),
        pltpu.VMEM((CHUNK, D), jnp.float32),
        pltpu.VMEM((CHUNK, D), jnp.float32),
        pltpu.VMEM((CHUNK, D), jnp.float32),
        pltpu.VMEM((CHUNK, L), jnp.float32),
        pltpu.VMEM((CHUNK, L), jnp.float32),
        pltpu.SemaphoreType.DMA,
        pltpu.SemaphoreType.DMA,
    ]

    def body(h_hbm, u_hbm, v_hbm, out_hbm,
             u_v, v_v, ur0, ur1, vr0, vr1, ob0, ob1, sem, sem_o):
        c = lax.axis_index("c")
        s = lax.axis_index("s")
        ur = [ur0, ur1]
        vr = [vr0, vr1]
        ob = [ob0, ob1]

        def run(ebase, cptc):
            eptc = cptc * CHUNK
            pltpu.sync_copy(u_hbm.at[pl.ds(ebase, eptc)],
                            u_v.at[pl.ds(0, eptc)])
            pltpu.sync_copy(v_hbm.at[pl.ds(ebase, eptc)],
                            v_v.at[pl.ds(0, eptc)])

            def issue(j, b):
                return (
                    pltpu.async_copy(
                        h_hbm.at[u_v.at[pl.ds(j * CHUNK, CHUNK)]], ur[b], sem),
                    pltpu.async_copy(
                        h_hbm.at[v_v.at[pl.ds(j * CHUNK, CHUNK)]], vr[b], sem),
                )

            pend = [None, None]
            pend_o = [None, None]
            pend[0] = issue(0, 0)
            for j in range(cptc):
                b = j & 1
                nb = b ^ 1
                gu, gv = pend[b]
                gu.wait()
                gv.wait()
                if j + 1 < cptc:
                    pend[nb] = issue(j + 1, nb)
                if pend_o[b] is not None:
                    pend_o[b].wait()
                    pend_o[b] = None
                urb = ur[b]
                vrb = vr[b]
                obb = ob[b]

                def edge(e, _):
                    acc = urb[e, pl.ds(0, L)] * vrb[e, pl.ds(0, L)]
                    for k in range(1, D // L):
                        acc = acc + (urb[e, pl.ds(k * L, L)]
                                     * vrb[e, pl.ds(k * L, L)])
                    obb[e, :] = acc
                    return 0

                lax.fori_loop(0, CHUNK, edge, 0)
                pend_o[b] = pltpu.async_copy(
                    obb, out_hbm.at[pl.ds(ebase + j * CHUNK, CHUNK)], sem_o)
            for b in range(2):
                if pend_o[b] is not None:
                    pend_o[b].wait()

        @pl.when(c == 0)
        def _run0():
            run(s * cpt0 * CHUNK, cpt0)

        @pl.when(c == 1)
        def _run1():
            run(NS * cpt0 * CHUNK + s * cpt1 * CHUNK, cpt1)

    mesh = plsc.VectorSubcoreMesh(core_axis_name="c", subcore_axis_name="s")
    fn = pl.kernel(
        body,
        out_type=jax.ShapeDtypeStruct((u1.shape[0], L), jnp.float32),
        mesh=mesh,
        scratch_types=scratch,
        compiler_params=_SC_PARAMS,
    )
    return fn(h2, u1, v1)


def _pad_edges(idx, per_tile_chunks, reshape2d):
    total = NW * per_tile_chunks * CHUNK
    pad = total - idx.shape[0]
    idx = jnp.concatenate(
        [idx, jnp.full((pad,), N_PAD - 1, jnp.int32)]) if pad else idx
    return idx.reshape(NW * per_tile_chunks, CHUNK) if reshape2d else idx


def kernel(x, edge_index, decode_edge_index,
           W_self0, W_neigh0, b0, W_self1, W_neigh1, b1):
    n_nodes = x.shape[0]
    n_edges = edge_index.shape[1]
    n_dec = decode_edge_index.shape[1]

    e_cpt = -(-n_edges // (NW * CHUNK * 8)) * 8
    d_cpt = -(-n_dec // (NW * CHUNK))

    x_p = jnp.pad(x, ((0, N_PAD - n_nodes), (0, 0)))
    src2d = _pad_edges(edge_index[0], e_cpt, True)
    dst2d = _pad_edges(edge_index[1], e_cpt, True)
    u1 = _pad_edges(decode_edge_index[0], d_cpt, False)
    v1 = _pad_edges(decode_edge_index[1], d_cpt, False)

    b0r = b0.reshape(1, D)
    b1r = b1.reshape(1, D)

    z0 = _matmul(x_p, W_neigh0)
    deg = _deg_sc(dst2d)
    agg0 = _segsum_sc(z0, src2d, dst2d, with_deg=True)
    h1, z1 = _fuse1(x_p, W_self0, b0r, agg0, deg, W_neigh1)
    agg1 = _segsum_sc(z1, src2d, dst2d, with_deg=False)
    h2 = _fuse2(h1, W_self1, b1r, agg1, deg)
    parts = _decode_sc(h2, u1, v1)
    logits = _reduce_partials(parts)
    return logits[:n_dec]


# speedup vs baseline: 1.3901x; 1.2037x over previous
import functools

import jax
import jax.numpy as jnp
from jax import lax
from jax.experimental import pallas as pl
from jax.experimental.pallas import tpu as pltpu
from jax.experimental.pallas import tpu_sc as plsc

_SC_PARAMS = pltpu.CompilerParams(use_tc_tiling_on_sc=False)

NC = 2
NS = 16
NW = NC * NS
L = 16

N_PAD = 10240
CHUNK = 128
ZROWS = N_PAD // NS
D = 128


def _mm_body(x_ref, w_ref, o_ref):
    o_ref[...] = jnp.dot(x_ref[...], w_ref[...],
                         preferred_element_type=jnp.float32)


def _matmul(x, w):
    n = x.shape[0]
    blk = 2048
    return pl.pallas_call(
        _mm_body,
        grid=(n // blk,),
        in_specs=[pl.BlockSpec((blk, D), lambda i: (i, 0)),
                  pl.BlockSpec((D, D), lambda i: (0, 0))],
        out_specs=pl.BlockSpec((blk, D), lambda i: (i, 0)),
        out_shape=jax.ShapeDtypeStruct((n, D), jnp.float32),
    )(x, w)


def _fuse1_body(x_ref, ws_ref, b_ref, agg_ref, deg_ref, wn_ref, h_ref, z_ref):
    deg = deg_ref[...][:, 0:1]
    inv = 1.0 / jnp.maximum(deg, 1.0)
    a = agg_ref[...]
    agg = (a[0] + a[1]) * inv
    h = jnp.dot(x_ref[...], ws_ref[...], preferred_element_type=jnp.float32)
    h = jnp.maximum(h + agg + b_ref[...], 0.0)
    h_ref[...] = h
    z_ref[...] = jnp.dot(h, wn_ref[...], preferred_element_type=jnp.float32)


def _fuse1(x, w_self, b, agg, deg, w_neigh_next):
    n = x.shape[0]
    blk = 2048
    return pl.pallas_call(
        _fuse1_body,
        grid=(n // blk,),
        in_specs=[pl.BlockSpec((blk, D), lambda i: (i, 0)),
                  pl.BlockSpec((D, D), lambda i: (0, 0)),
                  pl.BlockSpec((1, D), lambda i: (0, 0)),
                  pl.BlockSpec((NC, blk, D), lambda i: (0, i, 0)),
                  pl.BlockSpec((blk, L), lambda i: (i, 0)),
                  pl.BlockSpec((D, D), lambda i: (0, 0))],
        out_specs=[pl.BlockSpec((blk, D), lambda i: (i, 0)),
                   pl.BlockSpec((blk, D), lambda i: (i, 0))],
        out_shape=[jax.ShapeDtypeStruct((n, D), jnp.float32),
                   jax.ShapeDtypeStruct((n, D), jnp.float32)],
    )(x, w_self, b, agg, deg, w_neigh_next)


def _fuse2_body(x_ref, ws_ref, b_ref, agg_ref, deg_ref, h_ref):
    deg = deg_ref[...][:, 0:1]
    inv = 1.0 / jnp.maximum(deg, 1.0)
    a = agg_ref[...]
    agg = (a[0] + a[1]) * inv
    h = jnp.dot(x_ref[...], ws_ref[...], preferred_element_type=jnp.float32)
    h_ref[...] = h + agg + b_ref[...]


def _fuse2(x, w_self, b, agg, deg):
    n = x.shape[0]
    blk = 2048
    return pl.pallas_call(
        _fuse2_body,
        grid=(n // blk,),
        in_specs=[pl.BlockSpec((blk, D), lambda i: (i, 0)),
                  pl.BlockSpec((D, D), lambda i: (0, 0)),
                  pl.BlockSpec((1, D), lambda i: (0, 0)),
                  pl.BlockSpec((NC, blk, D), lambda i: (0, i, 0)),
                  pl.BlockSpec((blk, L), lambda i: (i, 0))],
        out_specs=pl.BlockSpec((blk, D), lambda i: (i, 0)),
        out_shape=jax.ShapeDtypeStruct((n, D), jnp.float32),
    )(x, w_self, b, agg, deg)


def _reduce_body(p_ref, o_ref):
    o_ref[...] = jnp.sum(p_ref[...], axis=1, keepdims=True)


def _reduce_partials(p):
    n = p.shape[0]
    blk = 4096
    return pl.pallas_call(
        _reduce_body,
        grid=(n // blk,),
        in_specs=[pl.BlockSpec((blk, L), lambda i: (i, 0))],
        out_specs=pl.BlockSpec((blk, 1), lambda i: (i, 0)),
        out_shape=jax.ShapeDtypeStruct((n, 1), jnp.float32),
    )(p)


def _zero_vmem_rows(ref, nrows, ncol16):
    zv = jnp.zeros((L,), jnp.float32)

    def row(r, _):
        for j in range(ncol16):
            ref[r, pl.ds(j * L, L)] = zv
        return 0

    lax.fori_loop(0, nrows, row, 0)


def _segsum_sc(z, src2d, dst2d, with_deg):
    del with_deg
    cpt0, cpt1 = 96, 64
    cpp = 8

    out_type = [jax.ShapeDtypeStruct((NC, N_PAD, D), jnp.float32)]
    scratch = [
        pltpu.VMEM((cpp, CHUNK), jnp.int32),
        pltpu.VMEM((cpp, CHUNK), jnp.int32),
        pltpu.VMEM((CHUNK, D), jnp.float32),
        pltpu.VMEM((CHUNK, D), jnp.float32),
        pltpu.VMEM_SHARED((N_PAD, D), jnp.float32),
        pltpu.SemaphoreType.DMA,
        pltpu.SemaphoreType.DMA,
    ]

    def body(z_hbm, src_hbm, dst_hbm, agg_out,
             src_v, dst_v, rows0, rows1, tab, sem_g, sem_s):
        c = lax.axis_index("c")
        s = lax.axis_index("s")
        rows = [rows0, rows1]

        _zero_vmem_rows(rows0, CHUNK, D // L)
        for k in range(ZROWS // CHUNK):
            pltpu.sync_copy(rows0,
                            tab.at[pl.ds(s * ZROWS + k * CHUNK, CHUNK)])

        plsc.subcore_barrier()

        def run(base, cptc):
            for p in range(cptc // cpp):
                off = pl.multiple_of(base + p * cpp, 8)
                pltpu.sync_copy(src_hbm.at[pl.ds(off, cpp)], src_v)
                pltpu.sync_copy(dst_hbm.at[pl.ds(off, cpp)], dst_v)
                pend_s = [None, None]
                pend_g = pltpu.async_copy(z_hbm.at[src_v.at[0]], rows[0],
                                          sem_g)
                for j in range(cpp):
                    b = j & 1
                    nb = b ^ 1
                    pend_g.wait()
                    if j + 1 < cpp:
                        if pend_s[nb] is not None:
                            pend_s[nb].wait()
                            pend_s[nb] = None
                        pend_g = pltpu.async_copy(
                            z_hbm.at[src_v.at[j + 1]], rows[nb], sem_g)
                    pend_s[b] = pltpu.async_copy(
                        rows[b], tab.at[dst_v.at[j]], sem_s, add=True)
                for b in range(2):
                    if pend_s[b] is not None:
                        pend_s[b].wait()

        @pl.when(c == 0)
        def _run0():
            run(s * cpt0, cpt0)

        @pl.when(c == 1)
        def _run1():
            run(NS * cpt0 + s * cpt1, cpt1)

        plsc.subcore_barrier()

        pltpu.sync_copy(tab.at[pl.ds(s * ZROWS, ZROWS)],
                        agg_out.at[c, pl.ds(s * ZROWS, ZROWS)])

    mesh = plsc.VectorSubcoreMesh(core_axis_name="c", subcore_axis_name="s")
    fn = pl.kernel(body, out_type=out_type, mesh=mesh, scratch_types=scratch,
                   compiler_params=_SC_PARAMS)
    return fn(z, src2d, dst2d)[0]


def _deg_sc(dst2d):
    cpt = dst2d.shape[0] // NS

    scratch = [
        pltpu.VMEM((cpt, CHUNK), jnp.int32),
        pltpu.VMEM((CHUNK, L), jnp.float32),
        pltpu.VMEM_SHARED((N_PAD, L), jnp.float32),
        pltpu.SemaphoreType.DMA,
    ]

    def body(dst_hbm, deg_out, dst_v, ones_v, degtab, sem):
        c = lax.axis_index("c")
        s = lax.axis_index("s")

        @pl.when(c == 0)
        def _run():
            pltpu.sync_copy(dst_hbm.at[pl.ds(s * cpt, cpt)], dst_v)
            _zero_vmem_rows(ones_v, CHUNK, 1)
            for k in range(ZROWS // CHUNK):
                pltpu.sync_copy(ones_v,
                                degtab.at[pl.ds(s * ZROWS + k * CHUNK, CHUNK)])

        plsc.subcore_barrier()

        @pl.when(c == 0)
        def _scatter():
            ones = jnp.full((L,), 1.0, jnp.float32)

            def fill(r, _):
                ones_v[r, :] = ones
                return 0

            lax.fori_loop(0, CHUNK, fill, 0)
            pend = []
            for j in range(cpt):
                if len(pend) >= 8:
                    pend.pop(0).wait()
                pend.append(pltpu.async_copy(
                    ones_v, degtab.at[dst_v.at[j]], sem, add=True))
            for d in pend:
                d.wait()

        plsc.subcore_barrier()

        @pl.when(c == 0)
        def _dump():
            pltpu.sync_copy(degtab.at[pl.ds(s * ZROWS, ZROWS)],
                            deg_out.at[pl.ds(s * ZROWS, ZROWS)])

    mesh = plsc.VectorSubcoreMesh(core_axis_name="c", subcore_axis_name="s")
    fn = pl.kernel(
        body,
        out_type=jax.ShapeDtypeStruct((N_PAD, L), jnp.float32),
        mesh=mesh,
        scratch_types=scratch,
        compiler_params=_SC_PARAMS,
    )
    return fn(dst2d)


def _decode_sc(h2, u1, v1):
    DC = 64
    ept = u1.shape[0] // NS
    cpt = ept // DC

    scratch = [
        pltpu.VMEM((ept,), jnp.int32),
        pltpu.VMEM((ept,), jnp.int32),
        pltpu.VMEM((DC, D), jnp.float32),
        pltpu.VMEM((DC, D), jnp.float32),
        pltpu.VMEM((DC, D), jnp.float32),
        pltpu.VMEM((DC, D), jnp.float32),
        pltpu.VMEM((DC, L), jnp.float32),
        pltpu.VMEM((DC, L), jnp.float32),
        pltpu.VMEM_SHARED((N_PAD, D), jnp.float32),
        pltpu.SemaphoreType.DMA,
        pltpu.SemaphoreType.DMA,
    ]

    def body(h_hbm, u_hbm, v_hbm, out_hbm,
             u_v, v_v, ur0, ur1, vr0, vr1, ob0, ob1, htab, sem, sem_o):
        c = lax.axis_index("c")
        s = lax.axis_index("s")
        ur = [ur0, ur1]
        vr = [vr0, vr1]
        ob = [ob0, ob1]

        @pl.when(c == 0)
        def _stage():
            pltpu.sync_copy(h_hbm.at[pl.ds(s * ZROWS, ZROWS)],
                            htab.at[pl.ds(s * ZROWS, ZROWS)])

        plsc.subcore_barrier()

        @pl.when(c == 0)
        def _run():
            ebase = s * ept
            pltpu.sync_copy(u_hbm.at[pl.ds(ebase, ept)], u_v)
            pltpu.sync_copy(v_hbm.at[pl.ds(ebase, ept)], v_v)

            def issue(j, b):
                return (
                    pltpu.async_copy(
                        htab.at[u_v.at[pl.ds(j * DC, DC)]], ur[b], sem),
                    pltpu.async_copy(
                        htab.at[v_v.at[pl.ds(j * DC, DC)]], vr[b], sem),
                )

            pend = [None, None]
            pend_o = [None, None]
            pend[0] = issue(0, 0)
            for j in range(cpt):
                b = j & 1
                nb = b ^ 1
                gu, gv = pend[b]
                gu.wait()
                gv.wait()
                if j + 1 < cpt:
                    pend[nb] = issue(j + 1, nb)
                if pend_o[b] is not None:
                    pend_o[b].wait()
                    pend_o[b] = None
                urb = ur[b]
                vrb = vr[b]
                obb = ob[b]

                def edge(e, _):
                    acc = urb[e, pl.ds(0, L)] * vrb[e, pl.ds(0, L)]
                    for k in range(1, D // L):
                        acc = acc + (urb[e, pl.ds(k * L, L)]
                                     * vrb[e, pl.ds(k * L, L)])
                    obb[e, :] = acc
                    return 0

                lax.fori_loop(0, DC, edge, 0)
                pend_o[b] = pltpu.async_copy(
                    obb, out_hbm.at[pl.ds(ebase + j * DC, DC)], sem_o)
            for b in range(2):
                if pend_o[b] is not None:
                    pend_o[b].wait()

    mesh = plsc.VectorSubcoreMesh(core_axis_name="c", subcore_axis_name="s")
    fn = pl.kernel(
        body,
        out_type=jax.ShapeDtypeStruct((u1.shape[0], L), jnp.float32),
        mesh=mesh,
        scratch_types=scratch,
        compiler_params=_SC_PARAMS,
    )
    return fn(h2, u1, v1)


def _pad_edges(idx, per_tile_chunks, reshape2d):
    total = NW * per_tile_chunks * CHUNK
    pad = total - idx.shape[0]
    idx = jnp.concatenate(
        [idx, jnp.full((pad,), N_PAD - 1, jnp.int32)]) if pad else idx
    return idx.reshape(NW * per_tile_chunks, CHUNK) if reshape2d else idx


def kernel(x, edge_index, decode_edge_index,
           W_self0, W_neigh0, b0, W_self1, W_neigh1, b1):
    n_nodes = x.shape[0]
    n_edges = edge_index.shape[1]
    n_dec = decode_edge_index.shape[1]

    e_cpt = -(-n_edges // (NW * CHUNK * 8)) * 8
    d_cpt = -(-n_dec // (NW * CHUNK))

    x_p = jnp.pad(x, ((0, N_PAD - n_nodes), (0, 0)))
    src2d = _pad_edges(edge_index[0], e_cpt, True)
    dst2d = _pad_edges(edge_index[1], e_cpt, True)
    u1 = _pad_edges(decode_edge_index[0], d_cpt, False)
    v1 = _pad_edges(decode_edge_index[1], d_cpt, False)

    b0r = b0.reshape(1, D)
    b1r = b1.reshape(1, D)

    z0 = _matmul(x_p, W_neigh0)
    deg = _deg_sc(dst2d)
    agg0 = _segsum_sc(z0, src2d, dst2d, with_deg=True)
    h1, z1 = _fuse1(x_p, W_self0, b0r, agg0, deg, W_neigh1)
    agg1 = _segsum_sc(z1, src2d, dst2d, with_deg=False)
    h2 = _fuse2(h1, W_self1, b1r, agg1, deg)
    parts = _decode_sc(h2, u1, v1)
    logits = _reduce_partials(parts)
    return logits[:n_dec]


# speedup vs baseline: 2.1826x; 1.5702x over previous
import functools

import jax
import jax.numpy as jnp
from jax import lax
from jax.experimental import pallas as pl
from jax.experimental.pallas import tpu as pltpu
from jax.experimental.pallas import tpu_sc as plsc

_SC_PARAMS = pltpu.CompilerParams(use_tc_tiling_on_sc=False)

NC = 2
NS = 16
NW = NC * NS
L = 16

N_PAD = 10240
CHUNK = 128
ZROWS = N_PAD // NS
D = 128


def _mm_body(x_ref, w_ref, o_ref):
    o_ref[...] = jnp.dot(x_ref[...], w_ref[...],
                         preferred_element_type=jnp.float32)


def _matmul(x, w):
    n = x.shape[0]
    blk = 2048
    return pl.pallas_call(
        _mm_body,
        grid=(n // blk,),
        in_specs=[pl.BlockSpec((blk, D), lambda i: (i, 0)),
                  pl.BlockSpec((D, D), lambda i: (0, 0))],
        out_specs=pl.BlockSpec((blk, D), lambda i: (i, 0)),
        out_shape=jax.ShapeDtypeStruct((n, D), jnp.float32),
    )(x, w)


def _fuse1_body(x_ref, ws_ref, b_ref, agg_ref, deg_ref, wn_ref, h_ref, z_ref):
    deg = deg_ref[...][:, 0:1]
    inv = 1.0 / jnp.maximum(deg, 1.0)
    a = agg_ref[...]
    agg = jnp.concatenate([a[0], a[1]], axis=-1) * inv
    h = jnp.dot(x_ref[...], ws_ref[...], preferred_element_type=jnp.float32)
    h = jnp.maximum(h + agg + b_ref[...], 0.0)
    h_ref[...] = h
    z_ref[...] = jnp.dot(h, wn_ref[...], preferred_element_type=jnp.float32)


def _fuse1(x, w_self, b, agg, deg, w_neigh_next):
    n = x.shape[0]
    blk = 2048
    return pl.pallas_call(
        _fuse1_body,
        grid=(n // blk,),
        in_specs=[pl.BlockSpec((blk, D), lambda i: (i, 0)),
                  pl.BlockSpec((D, D), lambda i: (0, 0)),
                  pl.BlockSpec((1, D), lambda i: (0, 0)),
                  pl.BlockSpec((NC, blk, D // NC), lambda i: (0, i, 0)),
                  pl.BlockSpec((blk, L), lambda i: (i, 0)),
                  pl.BlockSpec((D, D), lambda i: (0, 0))],
        out_specs=[pl.BlockSpec((blk, D), lambda i: (i, 0)),
                   pl.BlockSpec((blk, D), lambda i: (i, 0))],
        out_shape=[jax.ShapeDtypeStruct((n, D), jnp.float32),
                   jax.ShapeDtypeStruct((n, D), jnp.float32)],
    )(x, w_self, b, agg, deg, w_neigh_next)


def _fuse2_body(x_ref, ws_ref, b_ref, agg_ref, deg_ref, h_ref):
    deg = deg_ref[...][:, 0:1]
    inv = 1.0 / jnp.maximum(deg, 1.0)
    a = agg_ref[...]
    agg = jnp.concatenate([a[0], a[1]], axis=-1) * inv
    h = jnp.dot(x_ref[...], ws_ref[...], preferred_element_type=jnp.float32)
    h_ref[...] = h + agg + b_ref[...]


def _fuse2(x, w_self, b, agg, deg):
    n = x.shape[0]
    blk = 2048
    return pl.pallas_call(
        _fuse2_body,
        grid=(n // blk,),
        in_specs=[pl.BlockSpec((blk, D), lambda i: (i, 0)),
                  pl.BlockSpec((D, D), lambda i: (0, 0)),
                  pl.BlockSpec((1, D), lambda i: (0, 0)),
                  pl.BlockSpec((NC, blk, D // NC), lambda i: (0, i, 0)),
                  pl.BlockSpec((blk, L), lambda i: (i, 0))],
        out_specs=pl.BlockSpec((blk, D), lambda i: (i, 0)),
        out_shape=jax.ShapeDtypeStruct((n, D), jnp.float32),
    )(x, w_self, b, agg, deg)


def _reduce_body(p_ref, o_ref):
    o_ref[...] = jnp.sum(p_ref[...], axis=1, keepdims=True)


def _reduce_partials(p):
    n = p.shape[0]
    blk = 4096
    return pl.pallas_call(
        _reduce_body,
        grid=(n // blk,),
        in_specs=[pl.BlockSpec((blk, L), lambda i: (i, 0))],
        out_specs=pl.BlockSpec((blk, 1), lambda i: (i, 0)),
        out_shape=jax.ShapeDtypeStruct((n, 1), jnp.float32),
    )(p)


def _zero_vmem_rows(ref, nrows, ncol16):
    zv = jnp.zeros((L,), jnp.float32)

    def row(r, _):
        for j in range(ncol16):
            ref[r, pl.ds(j * L, L)] = zv
        return 0

    lax.fori_loop(0, nrows, row, 0)


def _segsum_sc(z, src2d, dst2d, with_deg):
    del with_deg
    CW = D // NC
    cpt = src2d.shape[0] // NS
    cpp = 8

    out_type = [jax.ShapeDtypeStruct((NC, N_PAD, CW), jnp.float32)]
    scratch = [
        pltpu.VMEM((cpp, CHUNK), jnp.int32),
        pltpu.VMEM((cpp, CHUNK), jnp.int32),
        pltpu.VMEM((CHUNK, CW), jnp.float32),
        pltpu.VMEM((CHUNK, CW), jnp.float32),
        pltpu.VMEM_SHARED((N_PAD, CW), jnp.float32),
        pltpu.VMEM_SHARED((N_PAD, CW), jnp.float32),
        pltpu.SemaphoreType.DMA,
        pltpu.SemaphoreType.DMA,
    ]

    def body(z_hbm, src_hbm, dst_hbm, agg_out,
             src_v, dst_v, rows0, rows1, ztab, acc, sem_g, sem_s):
        c = lax.axis_index("c")
        s = lax.axis_index("s")
        rows = [rows0, rows1]

        pltpu.sync_copy(
            z_hbm.at[pl.ds(s * ZROWS, ZROWS), pl.ds(c * CW, CW)],
            ztab.at[pl.ds(s * ZROWS, ZROWS)])
        _zero_vmem_rows(rows0, CHUNK, CW // L)
        for k in range(ZROWS // CHUNK):
            pltpu.sync_copy(rows0,
                            acc.at[pl.ds(s * ZROWS + k * CHUNK, CHUNK)])

        plsc.subcore_barrier()

        for p in range(cpt // cpp):
            off = pl.multiple_of(s * cpt + p * cpp, 8)
            pltpu.sync_copy(src_hbm.at[pl.ds(off, cpp)], src_v)
            pltpu.sync_copy(dst_hbm.at[pl.ds(off, cpp)], dst_v)
            pend_s = [None, None]
            pend_g = pltpu.async_copy(ztab.at[src_v.at[0]], rows[0], sem_g)
            for j in range(cpp):
                b = j & 1
                nb = b ^ 1
                pend_g.wait()
                if j + 1 < cpp:
                    if pend_s[nb] is not None:
                        pend_s[nb].wait()
                        pend_s[nb] = None
                    pend_g = pltpu.async_copy(
                        ztab.at[src_v.at[j + 1]], rows[nb], sem_g)
                pend_s[b] = pltpu.async_copy(
                    rows[b], acc.at[dst_v.at[j]], sem_s, add=True)
            for b in range(2):
                if pend_s[b] is not None:
                    pend_s[b].wait()

        plsc.subcore_barrier()

        pltpu.sync_copy(acc.at[pl.ds(s * ZROWS, ZROWS)],
                        agg_out.at[c, pl.ds(s * ZROWS, ZROWS)])

    mesh = plsc.VectorSubcoreMesh(core_axis_name="c", subcore_axis_name="s")
    fn = pl.kernel(body, out_type=out_type, mesh=mesh, scratch_types=scratch,
                   compiler_params=_SC_PARAMS)
    return fn(z, src2d, dst2d)[0]


def _deg_sc(dst2d):
    cpt = dst2d.shape[0] // NS

    scratch = [
        pltpu.VMEM((cpt, CHUNK), jnp.int32),
        pltpu.VMEM((CHUNK, L), jnp.float32),
        pltpu.VMEM_SHARED((N_PAD, L), jnp.float32),
        pltpu.SemaphoreType.DMA,
    ]

    def body(dst_hbm, deg_out, dst_v, ones_v, degtab, sem):
        c = lax.axis_index("c")
        s = lax.axis_index("s")

        @pl.when(c == 0)
        def _run():
            pltpu.sync_copy(dst_hbm.at[pl.ds(s * cpt, cpt)], dst_v)
            _zero_vmem_rows(ones_v, CHUNK, 1)
            for k in range(ZROWS // CHUNK):
                pltpu.sync_copy(ones_v,
                                degtab.at[pl.ds(s * ZROWS + k * CHUNK, CHUNK)])

        plsc.subcore_barrier()

        @pl.when(c == 0)
        def _scatter():
            ones = jnp.full((L,), 1.0, jnp.float32)

            def fill(r, _):
                ones_v[r, :] = ones
                return 0

            lax.fori_loop(0, CHUNK, fill, 0)
            pend = []
            for j in range(cpt):
                if len(pend) >= 8:
                    pend.pop(0).wait()
                pend.append(pltpu.async_copy(
                    ones_v, degtab.at[dst_v.at[j]], sem, add=True))
            for d in pend:
                d.wait()

        plsc.subcore_barrier()

        @pl.when(c == 0)
        def _dump():
            pltpu.sync_copy(degtab.at[pl.ds(s * ZROWS, ZROWS)],
                            deg_out.at[pl.ds(s * ZROWS, ZROWS)])

    mesh = plsc.VectorSubcoreMesh(core_axis_name="c", subcore_axis_name="s")
    fn = pl.kernel(
        body,
        out_type=jax.ShapeDtypeStruct((N_PAD, L), jnp.float32),
        mesh=mesh,
        scratch_types=scratch,
        compiler_params=_SC_PARAMS,
    )
    return fn(dst2d)


def _decode_sc(h2, u1, v1):
    DC = 64
    ept = u1.shape[0] // NS
    cpt = ept // DC

    scratch = [
        pltpu.VMEM((ept,), jnp.int32),
        pltpu.VMEM((ept,), jnp.int32),
        pltpu.VMEM((DC, D), jnp.float32),
        pltpu.VMEM((DC, D), jnp.float32),
        pltpu.VMEM((DC, D), jnp.float32),
        pltpu.VMEM((DC, D), jnp.float32),
        pltpu.VMEM((DC, L), jnp.float32),
        pltpu.VMEM((DC, L), jnp.float32),
        pltpu.VMEM_SHARED((N_PAD, D), jnp.float32),
        pltpu.SemaphoreType.DMA,
        pltpu.SemaphoreType.DMA,
    ]

    def body(h_hbm, u_hbm, v_hbm, out_hbm,
             u_v, v_v, ur0, ur1, vr0, vr1, ob0, ob1, htab, sem, sem_o):
        c = lax.axis_index("c")
        s = lax.axis_index("s")
        ur = [ur0, ur1]
        vr = [vr0, vr1]
        ob = [ob0, ob1]

        @pl.when(c == 0)
        def _stage():
            pltpu.sync_copy(h_hbm.at[pl.ds(s * ZROWS, ZROWS)],
                            htab.at[pl.ds(s * ZROWS, ZROWS)])

        plsc.subcore_barrier()

        @pl.when(c == 0)
        def _run():
            ebase = s * ept
            pltpu.sync_copy(u_hbm.at[pl.ds(ebase, ept)], u_v)
            pltpu.sync_copy(v_hbm.at[pl.ds(ebase, ept)], v_v)

            def issue(j, b):
                return (
                    pltpu.async_copy(
                        htab.at[u_v.at[pl.ds(j * DC, DC)]], ur[b], sem),
                    pltpu.async_copy(
                        htab.at[v_v.at[pl.ds(j * DC, DC)]], vr[b], sem),
                )

            pend = [None, None]
            pend_o = [None, None]
            pend[0] = issue(0, 0)
            for j in range(cpt):
                b = j & 1
                nb = b ^ 1
                gu, gv = pend[b]
                gu.wait()
                gv.wait()
                if j + 1 < cpt:
                    pend[nb] = issue(j + 1, nb)
                if pend_o[b] is not None:
                    pend_o[b].wait()
                    pend_o[b] = None
                urb = ur[b]
                vrb = vr[b]
                obb = ob[b]

                def edge(e, _):
                    acc = urb[e, pl.ds(0, L)] * vrb[e, pl.ds(0, L)]
                    for k in range(1, D // L):
                        acc = acc + (urb[e, pl.ds(k * L, L)]
                                     * vrb[e, pl.ds(k * L, L)])
                    obb[e, :] = acc
                    return 0

                lax.fori_loop(0, DC, edge, 0)
                pend_o[b] = pltpu.async_copy(
                    obb, out_hbm.at[pl.ds(ebase + j * DC, DC)], sem_o)
            for b in range(2):
                if pend_o[b] is not None:
                    pend_o[b].wait()

    mesh = plsc.VectorSubcoreMesh(core_axis_name="c", subcore_axis_name="s")
    fn = pl.kernel(
        body,
        out_type=jax.ShapeDtypeStruct((u1.shape[0], L), jnp.float32),
        mesh=mesh,
        scratch_types=scratch,
        compiler_params=_SC_PARAMS,
    )
    return fn(h2, u1, v1)


def _pad_edges(idx, per_tile_chunks, reshape2d):
    total = NW * per_tile_chunks * CHUNK
    pad = total - idx.shape[0]
    idx = jnp.concatenate(
        [idx, jnp.full((pad,), N_PAD - 1, jnp.int32)]) if pad else idx
    return idx.reshape(NW * per_tile_chunks, CHUNK) if reshape2d else idx


def kernel(x, edge_index, decode_edge_index,
           W_self0, W_neigh0, b0, W_self1, W_neigh1, b1):
    n_nodes = x.shape[0]
    n_edges = edge_index.shape[1]
    n_dec = decode_edge_index.shape[1]

    e_cpt = -(-n_edges // (NW * CHUNK * 8)) * 8
    d_cpt = -(-n_dec // (NW * CHUNK))

    x_p = jnp.pad(x, ((0, N_PAD - n_nodes), (0, 0)))
    src2d = _pad_edges(edge_index[0], e_cpt, True)
    dst2d = _pad_edges(edge_index[1], e_cpt, True)
    u1 = _pad_edges(decode_edge_index[0], d_cpt, False)
    v1 = _pad_edges(decode_edge_index[1], d_cpt, False)

    b0r = b0.reshape(1, D)
    b1r = b1.reshape(1, D)

    z0 = _matmul(x_p, W_neigh0)
    deg = _deg_sc(dst2d)
    agg0 = _segsum_sc(z0, src2d, dst2d, with_deg=True)
    h1, z1 = _fuse1(x_p, W_self0, b0r, agg0, deg, W_neigh1)
    agg1 = _segsum_sc(z1, src2d, dst2d, with_deg=False)
    h2 = _fuse2(h1, W_self1, b1r, agg1, deg)
    parts = _decode_sc(h2, u1, v1)
    logits = _reduce_partials(parts)
    return logits[:n_dec]


# speedup vs baseline: 2.3719x; 1.0867x over previous
import functools

import jax
import jax.numpy as jnp
from jax import lax
from jax.experimental import pallas as pl
from jax.experimental.pallas import tpu as pltpu
from jax.experimental.pallas import tpu_sc as plsc

_SC_PARAMS = pltpu.CompilerParams(use_tc_tiling_on_sc=False)

NC = 2
NS = 16
NW = NC * NS
L = 16

N_PAD = 10240
CHUNK = 128
ZROWS = N_PAD // NS
D = 128


def _mm_body(x_ref, w_ref, o_ref):
    o_ref[...] = jnp.dot(x_ref[...], w_ref[...],
                         preferred_element_type=jnp.float32)


def _matmul(x, w):
    n = x.shape[0]
    blk = 2048
    return pl.pallas_call(
        _mm_body,
        grid=(n // blk,),
        in_specs=[pl.BlockSpec((blk, D), lambda i: (i, 0)),
                  pl.BlockSpec((D, D), lambda i: (0, 0))],
        out_specs=pl.BlockSpec((blk, D), lambda i: (i, 0)),
        out_shape=jax.ShapeDtypeStruct((n, D), jnp.float32),
    )(x, w)


def _fuse1_body(x_ref, ws_ref, b_ref, agg_ref, deg_ref, wn_ref, h_ref, z_ref):
    deg = deg_ref[...][:, 0:1]
    inv = 1.0 / jnp.maximum(deg, 1.0)
    a = agg_ref[...]
    agg = jnp.concatenate([a[0], a[1]], axis=-1) * inv
    h = jnp.dot(x_ref[...], ws_ref[...], preferred_element_type=jnp.float32)
    h = jnp.maximum(h + agg + b_ref[...], 0.0)
    h_ref[...] = h
    z_ref[...] = jnp.dot(h, wn_ref[...], preferred_element_type=jnp.float32)


def _fuse1(x, w_self, b, agg, deg, w_neigh_next):
    n = x.shape[0]
    blk = 2048
    return pl.pallas_call(
        _fuse1_body,
        grid=(n // blk,),
        in_specs=[pl.BlockSpec((blk, D), lambda i: (i, 0)),
                  pl.BlockSpec((D, D), lambda i: (0, 0)),
                  pl.BlockSpec((1, D), lambda i: (0, 0)),
                  pl.BlockSpec((NC, blk, D // NC), lambda i: (0, i, 0)),
                  pl.BlockSpec((blk, L), lambda i: (i, 0)),
                  pl.BlockSpec((D, D), lambda i: (0, 0))],
        out_specs=[pl.BlockSpec((blk, D), lambda i: (i, 0)),
                   pl.BlockSpec((blk, D), lambda i: (i, 0))],
        out_shape=[jax.ShapeDtypeStruct((n, D), jnp.float32),
                   jax.ShapeDtypeStruct((n, D), jnp.float32)],
    )(x, w_self, b, agg, deg, w_neigh_next)


def _fuse2_body(x_ref, ws_ref, b_ref, agg_ref, deg_ref, h_ref):
    deg = deg_ref[...][:, 0:1]
    inv = 1.0 / jnp.maximum(deg, 1.0)
    a = agg_ref[...]
    agg = jnp.concatenate([a[0], a[1]], axis=-1) * inv
    h = jnp.dot(x_ref[...], ws_ref[...], preferred_element_type=jnp.float32)
    h_ref[...] = h + agg + b_ref[...]


def _fuse2(x, w_self, b, agg, deg):
    n = x.shape[0]
    blk = 2048
    return pl.pallas_call(
        _fuse2_body,
        grid=(n // blk,),
        in_specs=[pl.BlockSpec((blk, D), lambda i: (i, 0)),
                  pl.BlockSpec((D, D), lambda i: (0, 0)),
                  pl.BlockSpec((1, D), lambda i: (0, 0)),
                  pl.BlockSpec((NC, blk, D // NC), lambda i: (0, i, 0)),
                  pl.BlockSpec((blk, L), lambda i: (i, 0))],
        out_specs=pl.BlockSpec((blk, D), lambda i: (i, 0)),
        out_shape=jax.ShapeDtypeStruct((n, D), jnp.float32),
    )(x, w_self, b, agg, deg)


def _reduce_body(p_ref, o_ref):
    o_ref[...] = jnp.sum(p_ref[...], axis=1, keepdims=True)


def _reduce_partials(p):
    n = p.shape[0]
    blk = 4096
    return pl.pallas_call(
        _reduce_body,
        grid=(n // blk,),
        in_specs=[pl.BlockSpec((blk, L), lambda i: (i, 0))],
        out_specs=pl.BlockSpec((blk, 1), lambda i: (i, 0)),
        out_shape=jax.ShapeDtypeStruct((n, 1), jnp.float32),
    )(p)


def _zero_vmem_rows(ref, nrows, ncol16):
    zv = jnp.zeros((L,), jnp.float32)

    def row(r, _):
        for j in range(ncol16):
            ref[r, pl.ds(j * L, L)] = zv
        return 0

    lax.fori_loop(0, nrows, row, 0)


def _segsum_sc(z, src2d, dst2d, with_deg):
    del with_deg
    CW = D // NC
    cpt = src2d.shape[0] // NS
    cpp = 16

    out_type = [jax.ShapeDtypeStruct((NC, N_PAD, CW), jnp.float32)]
    scratch = [
        pltpu.VMEM((cpp, CHUNK), jnp.int32),
        pltpu.VMEM((cpp, CHUNK), jnp.int32),
        pltpu.VMEM((CHUNK, CW), jnp.float32),
        pltpu.VMEM((CHUNK, CW), jnp.float32),
        pltpu.VMEM((CHUNK, CW), jnp.float32),
        pltpu.VMEM((CHUNK, CW), jnp.float32),
        pltpu.VMEM_SHARED((N_PAD, CW), jnp.float32),
        pltpu.VMEM_SHARED((N_PAD, CW), jnp.float32),
        pltpu.SemaphoreType.DMA,
        pltpu.SemaphoreType.DMA,
    ]

    def body(z_hbm, src_hbm, dst_hbm, agg_out,
             src_v, dst_v, rows0, rows1, rows2, rows3, ztab, acc,
             sem_g, sem_s):
        c = lax.axis_index("c")
        s = lax.axis_index("s")
        rows = [rows0, rows1, rows2, rows3]

        pltpu.sync_copy(
            z_hbm.at[pl.ds(s * ZROWS, ZROWS), pl.ds(c * CW, CW)],
            ztab.at[pl.ds(s * ZROWS, ZROWS)])
        _zero_vmem_rows(rows0, CHUNK, CW // L)
        for k in range(ZROWS // CHUNK):
            pltpu.sync_copy(rows0,
                            acc.at[pl.ds(s * ZROWS + k * CHUNK, CHUNK)])

        plsc.subcore_barrier()

        for p in range(cpt // cpp):
            off = pl.multiple_of(s * cpt + p * cpp, 8)
            pltpu.sync_copy(src_hbm.at[pl.ds(off, cpp)], src_v)
            pltpu.sync_copy(dst_hbm.at[pl.ds(off, cpp)], dst_v)
            pend_g = [None] * 4
            pend_s = [None] * 4
            for j in range(min(3, cpp)):
                pend_g[j % 4] = pltpu.async_copy(
                    ztab.at[src_v.at[j]], rows[j % 4], sem_g)
            for j in range(cpp):
                b = j % 4
                pend_g[b].wait()
                pend_g[b] = None
                jn = j + 3
                if jn < cpp:
                    nb = jn % 4
                    if pend_s[nb] is not None:
                        pend_s[nb].wait()
                        pend_s[nb] = None
                    pend_g[nb] = pltpu.async_copy(
                        ztab.at[src_v.at[jn]], rows[nb], sem_g)
                pend_s[b] = pltpu.async_copy(
                    rows[b], acc.at[dst_v.at[j]], sem_s, add=True)
            for b in range(4):
                if pend_s[b] is not None:
                    pend_s[b].wait()

        plsc.subcore_barrier()

        pltpu.sync_copy(acc.at[pl.ds(s * ZROWS, ZROWS)],
                        agg_out.at[c, pl.ds(s * ZROWS, ZROWS)])

    mesh = plsc.VectorSubcoreMesh(core_axis_name="c", subcore_axis_name="s")
    fn = pl.kernel(body, out_type=out_type, mesh=mesh, scratch_types=scratch,
                   compiler_params=_SC_PARAMS)
    return fn(z, src2d, dst2d)[0]


def _deg_sc(dst2d):
    cpt = dst2d.shape[0] // NS

    scratch = [
        pltpu.VMEM((cpt, CHUNK), jnp.int32),
        pltpu.VMEM((CHUNK, L), jnp.float32),
        pltpu.VMEM_SHARED((N_PAD, L), jnp.float32),
        pltpu.SemaphoreType.DMA,
    ]

    def body(dst_hbm, deg_out, dst_v, ones_v, degtab, sem):
        c = lax.axis_index("c")
        s = lax.axis_index("s")

        @pl.when(c == 0)
        def _run():
            pltpu.sync_copy(dst_hbm.at[pl.ds(s * cpt, cpt)], dst_v)
            _zero_vmem_rows(ones_v, CHUNK, 1)
            for k in range(ZROWS // CHUNK):
                pltpu.sync_copy(ones_v,
                                degtab.at[pl.ds(s * ZROWS + k * CHUNK, CHUNK)])

        plsc.subcore_barrier()

        @pl.when(c == 0)
        def _scatter():
            ones = jnp.full((L,), 1.0, jnp.float32)

            def fill(r, _):
                ones_v[r, :] = ones
                return 0

            lax.fori_loop(0, CHUNK, fill, 0)
            pend = []
            for j in range(cpt):
                if len(pend) >= 8:
                    pend.pop(0).wait()
                pend.append(pltpu.async_copy(
                    ones_v, degtab.at[dst_v.at[j]], sem, add=True))
            for d in pend:
                d.wait()

        plsc.subcore_barrier()

        @pl.when(c == 0)
        def _dump():
            pltpu.sync_copy(degtab.at[pl.ds(s * ZROWS, ZROWS)],
                            deg_out.at[pl.ds(s * ZROWS, ZROWS)])

    mesh = plsc.VectorSubcoreMesh(core_axis_name="c", subcore_axis_name="s")
    fn = pl.kernel(
        body,
        out_type=jax.ShapeDtypeStruct((N_PAD, L), jnp.float32),
        mesh=mesh,
        scratch_types=scratch,
        compiler_params=_SC_PARAMS,
    )
    return fn(dst2d)


def _decode_sc(h2, u1, v1):
    DC = 64
    ept = u1.shape[0] // NS
    cpt = ept // DC

    scratch = [
        pltpu.VMEM((ept,), jnp.int32),
        pltpu.VMEM((ept,), jnp.int32),
        pltpu.VMEM((DC, D), jnp.float32),
        pltpu.VMEM((DC, D), jnp.float32),
        pltpu.VMEM((DC, D), jnp.float32),
        pltpu.VMEM((DC, D), jnp.float32),
        pltpu.VMEM((DC, L), jnp.float32),
        pltpu.VMEM((DC, L), jnp.float32),
        pltpu.VMEM_SHARED((N_PAD, D), jnp.float32),
        pltpu.SemaphoreType.DMA,
        pltpu.SemaphoreType.DMA,
    ]

    def body(h_hbm, u_hbm, v_hbm, out_hbm,
             u_v, v_v, ur0, ur1, vr0, vr1, ob0, ob1, htab, sem, sem_o):
        c = lax.axis_index("c")
        s = lax.axis_index("s")
        ur = [ur0, ur1]
        vr = [vr0, vr1]
        ob = [ob0, ob1]

        @pl.when(c == 0)
        def _stage():
            pltpu.sync_copy(h_hbm.at[pl.ds(s * ZROWS, ZROWS)],
                            htab.at[pl.ds(s * ZROWS, ZROWS)])

        plsc.subcore_barrier()

        @pl.when(c == 0)
        def _run():
            ebase = s * ept
            pltpu.sync_copy(u_hbm.at[pl.ds(ebase, ept)], u_v)
            pltpu.sync_copy(v_hbm.at[pl.ds(ebase, ept)], v_v)

            def issue(j, b):
                return (
                    pltpu.async_copy(
                        htab.at[u_v.at[pl.ds(j * DC, DC)]], ur[b], sem),
                    pltpu.async_copy(
                        htab.at[v_v.at[pl.ds(j * DC, DC)]], vr[b], sem),
                )

            pend = [None, None]
            pend_o = [None, None]
            pend[0] = issue(0, 0)
            for j in range(cpt):
                b = j & 1
                nb = b ^ 1
                gu, gv = pend[b]
                gu.wait()
                gv.wait()
                if j + 1 < cpt:
                    pend[nb] = issue(j + 1, nb)
                if pend_o[b] is not None:
                    pend_o[b].wait()
                    pend_o[b] = None
                urb = ur[b]
                vrb = vr[b]
                obb = ob[b]

                def edge(e, _):
                    acc = urb[e, pl.ds(0, L)] * vrb[e, pl.ds(0, L)]
                    for k in range(1, D // L):
                        acc = acc + (urb[e, pl.ds(k * L, L)]
                                     * vrb[e, pl.ds(k * L, L)])
                    obb[e, :] = acc
                    return 0

                lax.fori_loop(0, DC, edge, 0)
                pend_o[b] = pltpu.async_copy(
                    obb, out_hbm.at[pl.ds(ebase + j * DC, DC)], sem_o)
            for b in range(2):
                if pend_o[b] is not None:
                    pend_o[b].wait()

    mesh = plsc.VectorSubcoreMesh(core_axis_name="c", subcore_axis_name="s")
    fn = pl.kernel(
        body,
        out_type=jax.ShapeDtypeStruct((u1.shape[0], L), jnp.float32),
        mesh=mesh,
        scratch_types=scratch,
        compiler_params=_SC_PARAMS,
    )
    return fn(h2, u1, v1)


def _pad_edges(idx, per_tile_chunks, reshape2d):
    total = NW * per_tile_chunks * CHUNK
    pad = total - idx.shape[0]
    idx = jnp.concatenate(
        [idx, jnp.full((pad,), N_PAD - 1, jnp.int32)]) if pad else idx
    return idx.reshape(NW * per_tile_chunks, CHUNK) if reshape2d else idx


def kernel(x, edge_index, decode_edge_index,
           W_self0, W_neigh0, b0, W_self1, W_neigh1, b1):
    n_nodes = x.shape[0]
    n_edges = edge_index.shape[1]
    n_dec = decode_edge_index.shape[1]

    e_cpt = -(-n_edges // (NW * CHUNK * 8)) * 8
    d_cpt = -(-n_dec // (NW * CHUNK))

    x_p = jnp.pad(x, ((0, N_PAD - n_nodes), (0, 0)))
    src2d = _pad_edges(edge_index[0], e_cpt, True)
    dst2d = _pad_edges(edge_index[1], e_cpt, True)
    u1 = _pad_edges(decode_edge_index[0], d_cpt, False)
    v1 = _pad_edges(decode_edge_index[1], d_cpt, False)

    b0r = b0.reshape(1, D)
    b1r = b1.reshape(1, D)

    z0 = _matmul(x_p, W_neigh0)
    deg = _deg_sc(dst2d)
    agg0 = _segsum_sc(z0, src2d, dst2d, with_deg=True)
    h1, z1 = _fuse1(x_p, W_self0, b0r, agg0, deg, W_neigh1)
    agg1 = _segsum_sc(z1, src2d, dst2d, with_deg=False)
    h2 = _fuse2(h1, W_self1, b1r, agg1, deg)
    parts = _decode_sc(h2, u1, v1)
    logits = _reduce_partials(parts)
    return logits[:n_dec]
